# baseline (speedup 1.0000x reference)
import jax
import jax.numpy as jnp
from jax import lax
from jax.experimental import pallas as pl
from jax.experimental.pallas import tpu as pltpu
from jax.experimental.pallas import tpu_sc as plsc

N = 10000
NP = 10240
E = 320000
D = 128
NW = 32
EPW = E // NW
CHK = 125
NCH = EPW // CHK
NHALF = 2
HNCH = NCH // NHALF
NBUF = 2
SPR = NP // 16
WCH = 80
BLK = 1024


def _sc_mesh():
    return plsc.VectorSubcoreMesh(core_axis_name="c", subcore_axis_name="s",
                                  num_cores=2, num_subcores=16)


def _sc_rows_body(table_hbm, src_hbm, dst_hbm, out_hbm,
                  src_v, dst_v, b0, b1, acc_sh,
                  g0, g1, s0, s1):
    cid = lax.axis_index("c")
    sid = lax.axis_index("s")
    wid = cid * 16 + sid
    bufs = [b0, b1]
    gsem = [g0, g1]
    ssem = [s0, s1]

    def zr(r, _):
        def zc(c, _):
            b0[r, pl.ds(c * 16, 16)] = jnp.zeros((16,), jnp.float32)
            return 0
        return lax.fori_loop(0, D // 16, zc, 0)
    lax.fori_loop(0, WCH, zr, 0)

    def zs(z, _):
        pltpu.sync_copy(b0.at[pl.ds(0, WCH)],
                        acc_sh.at[pl.ds(sid * SPR + z * WCH, WCH)])
        return 0
    lax.fori_loop(0, SPR // WCH, zs, 0)

    plsc.subcore_barrier()

    def half(h, _):
        pltpu.sync_copy(src_hbm.at[wid, h], src_v)
        pltpu.sync_copy(dst_hbm.at[wid, h], dst_v)
        for q in range(NBUF):
            pltpu.async_copy(table_hbm.at[src_v.at[q]], bufs[q], gsem[q])

        def outer(jj, _):
            base = jj * NBUF
            for q in range(NBUF):
                c = base + q
                pltpu.make_async_copy(table_hbm.at[src_v.at[c]], bufs[q],
                                      gsem[q]).wait()
                pltpu.async_copy(bufs[q], acc_sh.at[dst_v.at[c]], ssem[q],
                                 add=True)
            for q in range(NBUF):
                c = base + q
                pltpu.make_async_copy(bufs[q], acc_sh.at[dst_v.at[c]],
                                      ssem[q]).wait()
                pltpu.async_copy(table_hbm.at[src_v.at[c + NBUF]], bufs[q],
                                 gsem[q])
            return 0
        lax.fori_loop(0, HNCH // NBUF - 1, outer, 0)

        base = HNCH - NBUF
        for q in range(NBUF):
            c = base + q
            pltpu.make_async_copy(table_hbm.at[src_v.at[c]], bufs[q],
                                  gsem[q]).wait()
            pltpu.async_copy(bufs[q], acc_sh.at[dst_v.at[c]], ssem[q],
                             add=True)
        for q in range(NBUF):
            c = base + q
            pltpu.make_async_copy(bufs[q], acc_sh.at[dst_v.at[c]],
                                  ssem[q]).wait()
        return 0
    lax.fori_loop(0, NHALF, half, 0)
    plsc.subcore_barrier()

    def wout(z, _):
        base = sid * SPR + z * WCH
        wb = b0.at[pl.ds(0, WCH)]
        pltpu.sync_copy(acc_sh.at[pl.ds(base, WCH)], wb)
        pltpu.sync_copy(wb, out_hbm.at[cid].at[pl.ds(base, WCH)])
        return 0
    lax.fori_loop(0, SPR // WCH, wout, 0)


def _sc_rows(table, src3, dst3):
    f = pl.kernel(
        _sc_rows_body,
        out_type=jax.ShapeDtypeStruct((2, NP, D), jnp.float32),
        mesh=_sc_mesh(),
        scratch_types=[
            pltpu.VMEM((HNCH, CHK), jnp.int32),
            pltpu.VMEM((HNCH, CHK), jnp.int32),
            pltpu.VMEM((CHK, D), jnp.float32),
            pltpu.VMEM((CHK, D), jnp.float32),
            pltpu.VMEM_SHARED((NP, D), jnp.float32),
            pltpu.SemaphoreType.DMA,
            pltpu.SemaphoreType.DMA,
            pltpu.SemaphoreType.DMA,
            pltpu.SemaphoreType.DMA,
        ],
    )
    return f(table, src3, dst3)


def _sc_scalar_body(vals_hbm, src_hbm, dst_hbm, out_hbm,
                    tab_v, src_v, dst_v, acc_v):
    cid = lax.axis_index("c")
    sid = lax.axis_index("s")
    wid = cid * 16 + sid
    pltpu.sync_copy(vals_hbm, tab_v)
    pltpu.sync_copy(src_hbm.at[wid], src_v)
    pltpu.sync_copy(dst_hbm.at[wid], dst_v)

    def z(i, _):
        acc_v[pl.ds(i * 16, 16)] = jnp.zeros((16,), jnp.float32)
        return 0
    lax.fori_loop(0, NP // 16, z, 0)

    def step(i, _):
        s = src_v[0, pl.ds(i * 16, 16)]
        t = dst_v[0, pl.ds(i * 16, 16)]
        v = plsc.load_gather(tab_v, [s])
        plsc.addupdate_scatter(acc_v, [t], v)
        return 0
    lax.fori_loop(0, EPW // 16, step, 0)
    pltpu.sync_copy(acc_v, out_hbm.at[wid, 0])


def _sc_scalar(vals, src2, dst2):
    f = pl.kernel(
        _sc_scalar_body,
        out_type=jax.ShapeDtypeStruct((NW, 1, NP), jnp.float32),
        mesh=_sc_mesh(),
        compiler_params=pltpu.CompilerParams(needs_layout_passes=False),
        scratch_types=[
            pltpu.VMEM((NP,), jnp.float32),
            pltpu.VMEM((1, EPW), jnp.int32),
            pltpu.VMEM((1, EPW), jnp.int32),
            pltpu.VMEM((NP,), jnp.float32),
        ],
    )
    return f(vals, src2, dst2).reshape(NW, NP).T


def _tc_conv_call(parts, cntT, xin, Wl, bl, Wr, mask=None, Wp=None,
                  mask_h=False, mask_r=False):
    has_mask = mask is not None
    has_pool = Wp is not None

    def body(*refs):
        i = 0
        parts_r = refs[i]; i += 1
        cnt_r = refs[i]; i += 1
        x_r = refs[i]; i += 1
        wl_r = refs[i]; i += 1
        bl_r = refs[i]; i += 1
        wr_r = refs[i]; i += 1
        mask_ref = refs[i] if has_mask else None
        i += 1 if has_mask else 0
        wp_r = refs[i] if has_pool else None
        i += 1 if has_pool else 0
        h_out = refs[i]; i += 1
        pool_out = refs[i] if has_pool else None

        A = parts_r[0] + parts_r[1]
        cnt = jnp.sum(cnt_r[...], axis=1, keepdims=True)
        mean = A * (1.0 / jnp.maximum(cnt, 1.0))
        h = jnp.dot(mean, wl_r[...], preferred_element_type=jnp.float32)
        h = h + bl_r[...] + jnp.dot(x_r[...], wr_r[...],
                                    preferred_element_type=jnp.float32)
        h = jnp.maximum(h, 0.0)
        if mask_h:
            h = h * mask_ref[...]
        h_out[...] = h
        if has_pool:
            p = jnp.dot(h, wp_r[...], preferred_element_type=jnp.float32)
            if mask_r:
                p = p * jnp.concatenate(
                    [mask_ref[...], jnp.ones_like(mask_ref[...])], axis=1)
            pool_out[...] = p

    grid = NP // BLK
    in_specs = [
        pl.BlockSpec((2, BLK, D), lambda i: (0, i, 0)),
        pl.BlockSpec((BLK, NW), lambda i: (i, 0)),
        pl.BlockSpec((BLK, D), lambda i: (i, 0)),
        pl.BlockSpec((D, D), lambda i: (0, 0)),
        pl.BlockSpec((1, D), lambda i: (0, 0)),
        pl.BlockSpec((D, D), lambda i: (0, 0)),
    ]
    args = [parts, cntT, xin, Wl, bl, Wr]
    if has_mask:
        in_specs.append(pl.BlockSpec((BLK, 1), lambda i: (i, 0)))
        args.append(mask)
    if has_pool:
        in_specs.append(pl.BlockSpec((D, 2), lambda i: (0, 0)))
        args.append(Wp)
    out_shape = [jax.ShapeDtypeStruct((NP, D), jnp.float32)]
    out_specs = [pl.BlockSpec((BLK, D), lambda i: (i, 0))]
    if has_pool:
        out_shape.append(jax.ShapeDtypeStruct((NP, 2), jnp.float32))
        out_specs.append(pl.BlockSpec((BLK, 2), lambda i: (i, 0)))

    res = pl.pallas_call(
        body,
        grid=(grid,),
        in_specs=in_specs,
        out_specs=out_specs,
        out_shape=out_shape,
    )(*args)
    return res if has_pool else res[0]


def _topk_mask(score, idx, k):
    key = lax.bitcast_convert_type(score, jnp.int32)
    ukey = (jnp.where(key < 0, jnp.int32(0x7FFFFFFF) ^ key, key)
            .astype(jnp.uint32) + jnp.uint32(2 ** 31))

    def tstep(b, t):
        cand = t | (jnp.uint32(1) << (31 - b))
        c = jnp.sum((ukey >= cand).astype(jnp.int32))
        return jnp.where(c >= k, cand, t)
    t = lax.fori_loop(0, 32, tstep, jnp.uint32(0))

    cnt_gt = jnp.sum((ukey > t).astype(jnp.int32))
    need = k - cnt_gt

    def mstep(b, m):
        cand = m | (jnp.int32(1) << (13 - b))
        c = jnp.sum(((ukey == t) & (idx < cand)).astype(jnp.int32))
        return jnp.where(c < need, cand, m)
    m = lax.fori_loop(0, 14, mstep, jnp.int32(0))

    act = (ukey > t) | ((ukey == t) & (idx <= m))
    return act.astype(jnp.float32)


def _tc_pool1_call(spartsT, root, brel, h2, idxcol, k):
    def body(sp_r, root_r, brel_r, h2_r, idx_r, act_out, hp_out):
        S = jnp.sum(sp_r[...], axis=1, keepdims=True)
        score = S + brel_r[0, 0] + root_r[...]
        score = jnp.where(idx_r[...] < N, score, -jnp.inf)
        act = _topk_mask(score, idx_r[...], k)
        act_out[...] = act
        hp_out[...] = h2_r[...] * (jnp.tanh(score) * act)

    return pl.pallas_call(
        body,
        out_shape=[jax.ShapeDtypeStruct((NP, 1), jnp.float32),
                   jax.ShapeDtypeStruct((NP, D), jnp.float32)],
    )(spartsT, root, brel, h2, idxcol)


def _tc_pool2_call(spartsT, root, brel, act1, h4, idxcol,
                   f1W, f1b, f2W, f2b, k):
    def body(sp_r, root_r, brel_r, act1_r, h4_r, idx_r,
             f1w_r, f1b_r, f2w_r, f2b_r, out_r):
        S = jnp.sum(sp_r[...], axis=1, keepdims=True)
        score = S + brel_r[0, 0] + root_r[...]
        score = jnp.where(act1_r[...] > 0, score, -jnp.inf)
        act2 = _topk_mask(score, idx_r[...], k)
        gate = jnp.tanh(score) * act2
        g = jnp.sum(h4_r[...] * gate, axis=0, keepdims=True) / float(k)
        g = jnp.dot(g, f1w_r[...], preferred_element_type=jnp.float32)
        g = jnp.maximum(g + f1b_r[...], 0.0)
        o = jnp.dot(g, f2w_r[...], preferred_element_type=jnp.float32)
        o = o + f2b_r[...]
        omax = jnp.max(o)
        lse = jnp.log(jnp.sum(jnp.exp(o - omax))) + omax
        out_r[...] = o - lse

    return pl.pallas_call(
        body,
        out_shape=jax.ShapeDtypeStruct((1, 2), jnp.float32),
    )(spartsT, root, brel, act1, h4, idxcol, f1W, f1b, f2W, f2b)


def kernel(x, edge_index, batch,
           conv1_Wl, conv1_bl, conv1_Wr, conv2_Wl, conv2_bl, conv2_Wr,
           conv3_Wl, conv3_bl, conv3_Wr, conv4_Wl, conv4_bl, conv4_Wr,
           pool1_Wrel, pool1_brel, pool1_Wroot,
           pool2_Wrel, pool2_brel, pool2_Wroot,
           fc1_W, fc1_b, fc2_W, fc2_b):
    src = edge_index[0]
    dst = edge_index[1]
    src3 = src.reshape(NW, NHALF, HNCH, CHK)
    dst3 = dst.reshape(NW, NHALF, HNCH, CHK)
    src2 = src.reshape(NW, 1, EPW)
    dst2 = dst.reshape(NW, 1, EPW)
    ones_n = jnp.ones((NP,), jnp.float32)
    idxcol = jnp.arange(NP, dtype=jnp.int32).reshape(NP, 1)
    xp = jnp.pad(x, ((0, NP - N), (0, 0)))
    b1 = conv1_bl.reshape(1, D)
    b2 = conv2_bl.reshape(1, D)
    b3 = conv3_bl.reshape(1, D)
    b4 = conv4_bl.reshape(1, D)
    Wp1 = jnp.concatenate([pool1_Wrel, pool1_Wroot], axis=1)
    Wp2 = jnp.concatenate([pool2_Wrel, pool2_Wroot], axis=1)

    cnt1T = _sc_scalar(ones_n, src2, dst2)
    A1 = _sc_rows(x, src3, dst3)
    h1 = _tc_conv_call(A1, cnt1T, xp, conv1_Wl, b1, conv1_Wr)

    A2 = _sc_rows(h1, src3, dst3)
    h2, pools1 = _tc_conv_call(A2, cnt1T, h1, conv2_Wl, b2, conv2_Wr, Wp=Wp1)
    r1 = pools1[:, 0]
    root1 = pools1[:, 1:2]

    S1T = _sc_scalar(r1, src2, dst2)
    act1, h2p = _tc_pool1_call(S1T, root1, pool1_brel.reshape(1, 1), h2,
                               idxcol, k=8000)

    cnt3T = _sc_scalar(act1.reshape(NP), src2, dst2)
    A3 = _sc_rows(h2p, src3, dst3)
    h3m = _tc_conv_call(A3, cnt3T, h2p, conv3_Wl, b3, conv3_Wr,
                        mask=act1, mask_h=True)

    A4 = _sc_rows(h3m, src3, dst3)
    h4, pools2 = _tc_conv_call(A4, cnt3T, h3m, conv4_Wl, b4, conv4_Wr,
                               mask=act1, Wp=Wp2, mask_r=True)
    r2m = pools2[:, 0]
    root2 = pools2[:, 1:2]

    S2T = _sc_scalar(r2m, src2, dst2)
    out = _tc_pool2_call(S2T, root2, pool2_brel.reshape(1, 1), act1, h4,
                         idxcol, fc1_W, fc1_b.reshape(1, 64),
                         fc2_W, fc2_b.reshape(1, 2), k=6400)
    return out.reshape(2)


# speedup vs baseline: 1.1915x; 1.1915x over previous
import jax
import jax.numpy as jnp
from jax import lax
from jax.experimental import pallas as pl
from jax.experimental.pallas import tpu as pltpu
from jax.experimental.pallas import tpu_sc as plsc

N = 10000
NP = 10240
E = 320000
D = 128
NW = 32
EPW = E // NW
CHK = 125
NCH = EPW // CHK
NHALF = 2
HNCH = NCH // NHALF
NBUF = 2
SPR = NP // 16
WCH = 80
BLK = 1024


def _sc_mesh():
    return plsc.VectorSubcoreMesh(core_axis_name="c", subcore_axis_name="s",
                                  num_cores=2, num_subcores=16)


def _sc_rows_body(table_hbm, src_hbm, dst_hbm, out_hbm,
                  src_v, dst_v, b0, b1, acc_sh, g0, g1):
    cid = lax.axis_index("c")
    sid = lax.axis_index("s")
    wid = cid * 16 + sid

    def zr(r, _):
        def zc(c, _):
            b0[r, pl.ds(c * 16, 16)] = jnp.zeros((16,), jnp.float32)
            return 0
        return lax.fori_loop(0, D // 16, zc, 0)
    lax.fori_loop(0, WCH, zr, 0)

    def zs(z, _):
        pltpu.sync_copy(b0.at[pl.ds(0, WCH)],
                        acc_sh.at[pl.ds(sid * SPR + z * WCH, WCH)])
        return 0
    lax.fori_loop(0, SPR // WCH, zs, 0)

    plsc.subcore_barrier()

    def half(h, _):
        pltpu.sync_copy(src_hbm.at[wid, h], src_v)
        pltpu.sync_copy(dst_hbm.at[wid, h], dst_v)
        pltpu.async_copy(table_hbm.at[src_v.at[0]], b0, g0)
        pltpu.async_copy(table_hbm.at[src_v.at[1]], b1, g1)

        def step(jj, _):
            j = 2 * jj
            pltpu.make_async_copy(table_hbm.at[src_v.at[j]], b0, g0).wait()
            pltpu.sync_copy(b0, acc_sh.at[dst_v.at[j]], add=True)
            pltpu.async_copy(table_hbm.at[src_v.at[j + 2]], b0, g0)
            pltpu.make_async_copy(table_hbm.at[src_v.at[j + 1]], b1,
                                  g1).wait()
            pltpu.sync_copy(b1, acc_sh.at[dst_v.at[j + 1]], add=True)
            pltpu.async_copy(table_hbm.at[src_v.at[j + 3]], b1, g1)
            return 0
        lax.fori_loop(0, HNCH // 2 - 1, step, 0)

        j = HNCH - 2
        pltpu.make_async_copy(table_hbm.at[src_v.at[j]], b0, g0).wait()
        pltpu.sync_copy(b0, acc_sh.at[dst_v.at[j]], add=True)
        pltpu.make_async_copy(table_hbm.at[src_v.at[j + 1]], b1, g1).wait()
        pltpu.sync_copy(b1, acc_sh.at[dst_v.at[j + 1]], add=True)
        return 0
    lax.fori_loop(0, NHALF, half, 0)
    plsc.subcore_barrier()

    def wout(z, _):
        base = sid * SPR + z * WCH
        wb = b0.at[pl.ds(0, WCH)]
        pltpu.sync_copy(acc_sh.at[pl.ds(base, WCH)], wb)
        pltpu.sync_copy(wb, out_hbm.at[cid].at[pl.ds(base, WCH)])
        return 0
    lax.fori_loop(0, SPR // WCH, wout, 0)


def _sc_rows(table, src3, dst3):
    f = pl.kernel(
        _sc_rows_body,
        out_type=jax.ShapeDtypeStruct((2, NP, D), jnp.float32),
        mesh=_sc_mesh(),
        scratch_types=[
            pltpu.VMEM((HNCH, CHK), jnp.int32),
            pltpu.VMEM((HNCH, CHK), jnp.int32),
            pltpu.VMEM((CHK, D), jnp.float32),
            pltpu.VMEM((CHK, D), jnp.float32),
            pltpu.VMEM_SHARED((NP, D), jnp.float32),
            pltpu.SemaphoreType.DMA,
            pltpu.SemaphoreType.DMA,
        ],
    )
    return f(table, src3, dst3)


def _sc_scalar_body(vals_hbm, src_hbm, dst_hbm, out_hbm,
                    tab_v, src_v, dst_v, acc_v):
    cid = lax.axis_index("c")
    sid = lax.axis_index("s")
    wid = cid * 16 + sid
    pltpu.sync_copy(vals_hbm, tab_v)
    pltpu.sync_copy(src_hbm.at[wid], src_v)
    pltpu.sync_copy(dst_hbm.at[wid], dst_v)

    def z(i, _):
        acc_v[pl.ds(i * 16, 16)] = jnp.zeros((16,), jnp.float32)
        return 0
    lax.fori_loop(0, NP // 16, z, 0)

    def step(i, _):
        s = src_v[0, pl.ds(i * 16, 16)]
        t = dst_v[0, pl.ds(i * 16, 16)]
        v = plsc.load_gather(tab_v, [s])
        plsc.addupdate_scatter(acc_v, [t], v)
        return 0
    lax.fori_loop(0, EPW // 16, step, 0)
    pltpu.sync_copy(acc_v, out_hbm.at[wid, 0])


def _sc_scalar(vals, src2, dst2):
    f = pl.kernel(
        _sc_scalar_body,
        out_type=jax.ShapeDtypeStruct((NW, 1, NP), jnp.float32),
        mesh=_sc_mesh(),
        compiler_params=pltpu.CompilerParams(needs_layout_passes=False),
        scratch_types=[
            pltpu.VMEM((NP,), jnp.float32),
            pltpu.VMEM((1, EPW), jnp.int32),
            pltpu.VMEM((1, EPW), jnp.int32),
            pltpu.VMEM((NP,), jnp.float32),
        ],
    )
    return f(vals, src2, dst2).reshape(NW, NP).T


def _tc_conv_call(parts, cntT, xin, Wl, bl, Wr, mask=None, Wp=None,
                  mask_h=False, mask_r=False):
    has_mask = mask is not None
    has_pool = Wp is not None

    def body(*refs):
        i = 0
        parts_r = refs[i]; i += 1
        cnt_r = refs[i]; i += 1
        x_r = refs[i]; i += 1
        wl_r = refs[i]; i += 1
        bl_r = refs[i]; i += 1
        wr_r = refs[i]; i += 1
        mask_ref = refs[i] if has_mask else None
        i += 1 if has_mask else 0
        wp_r = refs[i] if has_pool else None
        i += 1 if has_pool else 0
        h_out = refs[i]; i += 1
        pool_out = refs[i] if has_pool else None

        A = parts_r[0] + parts_r[1]
        cnt = jnp.sum(cnt_r[...], axis=1, keepdims=True)
        mean = A * (1.0 / jnp.maximum(cnt, 1.0))
        h = jnp.dot(mean, wl_r[...], preferred_element_type=jnp.float32)
        h = h + bl_r[...] + jnp.dot(x_r[...], wr_r[...],
                                    preferred_element_type=jnp.float32)
        h = jnp.maximum(h, 0.0)
        if mask_h:
            h = h * mask_ref[...]
        h_out[...] = h
        if has_pool:
            p = jnp.dot(h, wp_r[...], preferred_element_type=jnp.float32)
            if mask_r:
                p = p * jnp.concatenate(
                    [mask_ref[...], jnp.ones_like(mask_ref[...])], axis=1)
            pool_out[...] = p

    grid = NP // BLK
    in_specs = [
        pl.BlockSpec((2, BLK, D), lambda i: (0, i, 0)),
        pl.BlockSpec((BLK, NW), lambda i: (i, 0)),
        pl.BlockSpec((BLK, D), lambda i: (i, 0)),
        pl.BlockSpec((D, D), lambda i: (0, 0)),
        pl.BlockSpec((1, D), lambda i: (0, 0)),
        pl.BlockSpec((D, D), lambda i: (0, 0)),
    ]
    args = [parts, cntT, xin, Wl, bl, Wr]
    if has_mask:
        in_specs.append(pl.BlockSpec((BLK, 1), lambda i: (i, 0)))
        args.append(mask)
    if has_pool:
        in_specs.append(pl.BlockSpec((D, 2), lambda i: (0, 0)))
        args.append(Wp)
    out_shape = [jax.ShapeDtypeStruct((NP, D), jnp.float32)]
    out_specs = [pl.BlockSpec((BLK, D), lambda i: (i, 0))]
    if has_pool:
        out_shape.append(jax.ShapeDtypeStruct((NP, 2), jnp.float32))
        out_specs.append(pl.BlockSpec((BLK, 2), lambda i: (i, 0)))

    res = pl.pallas_call(
        body,
        grid=(grid,),
        in_specs=in_specs,
        out_specs=out_specs,
        out_shape=out_shape,
    )(*args)
    return res if has_pool else res[0]


def _topk_mask(score, idx, k):
    key = lax.bitcast_convert_type(score, jnp.int32)
    ukey = (jnp.where(key < 0, jnp.int32(0x7FFFFFFF) ^ key, key)
            .astype(jnp.uint32) + jnp.uint32(2 ** 31))

    def tstep(b, t):
        cand = t | (jnp.uint32(1) << (31 - b))
        c = jnp.sum((ukey >= cand).astype(jnp.int32))
        return jnp.where(c >= k, cand, t)
    t = lax.fori_loop(0, 32, tstep, jnp.uint32(0))

    cnt_gt = jnp.sum((ukey > t).astype(jnp.int32))
    need = k - cnt_gt

    def mstep(b, m):
        cand = m | (jnp.int32(1) << (13 - b))
        c = jnp.sum(((ukey == t) & (idx < cand)).astype(jnp.int32))
        return jnp.where(c < need, cand, m)
    m = lax.fori_loop(0, 14, mstep, jnp.int32(0))

    act = (ukey > t) | ((ukey == t) & (idx <= m))
    return act.astype(jnp.float32)


def _tc_pool1_call(spartsT, root, brel, h2, idxcol, k):
    def body(sp_r, root_r, brel_r, h2_r, idx_r, act_out, hp_out):
        S = jnp.sum(sp_r[...], axis=1, keepdims=True)
        score = S + brel_r[0, 0] + root_r[...]
        score = jnp.where(idx_r[...] < N, score, -jnp.inf)
        act = _topk_mask(score, idx_r[...], k)
        act_out[...] = act
        hp_out[...] = h2_r[...] * (jnp.tanh(score) * act)

    return pl.pallas_call(
        body,
        out_shape=[jax.ShapeDtypeStruct((NP, 1), jnp.float32),
                   jax.ShapeDtypeStruct((NP, D), jnp.float32)],
    )(spartsT, root, brel, h2, idxcol)


def _tc_pool2_call(spartsT, root, brel, act1, h4, idxcol,
                   f1W, f1b, f2W, f2b, k):
    def body(sp_r, root_r, brel_r, act1_r, h4_r, idx_r,
             f1w_r, f1b_r, f2w_r, f2b_r, out_r):
        S = jnp.sum(sp_r[...], axis=1, keepdims=True)
        score = S + brel_r[0, 0] + root_r[...]
        score = jnp.where(act1_r[...] > 0, score, -jnp.inf)
        act2 = _topk_mask(score, idx_r[...], k)
        gate = jnp.tanh(score) * act2
        g = jnp.sum(h4_r[...] * gate, axis=0, keepdims=True) / float(k)
        g = jnp.dot(g, f1w_r[...], preferred_element_type=jnp.float32)
        g = jnp.maximum(g + f1b_r[...], 0.0)
        o = jnp.dot(g, f2w_r[...], preferred_element_type=jnp.float32)
        o = o + f2b_r[...]
        omax = jnp.max(o)
        lse = jnp.log(jnp.sum(jnp.exp(o - omax))) + omax
        out_r[...] = o - lse

    return pl.pallas_call(
        body,
        out_shape=jax.ShapeDtypeStruct((1, 2), jnp.float32),
    )(spartsT, root, brel, act1, h4, idxcol, f1W, f1b, f2W, f2b)


def kernel(x, edge_index, batch,
           conv1_Wl, conv1_bl, conv1_Wr, conv2_Wl, conv2_bl, conv2_Wr,
           conv3_Wl, conv3_bl, conv3_Wr, conv4_Wl, conv4_bl, conv4_Wr,
           pool1_Wrel, pool1_brel, pool1_Wroot,
           pool2_Wrel, pool2_brel, pool2_Wroot,
           fc1_W, fc1_b, fc2_W, fc2_b):
    src = edge_index[0]
    dst = edge_index[1]
    src3 = src.reshape(NW, NHALF, HNCH, CHK)
    dst3 = dst.reshape(NW, NHALF, HNCH, CHK)
    src2 = src.reshape(NW, 1, EPW)
    dst2 = dst.reshape(NW, 1, EPW)
    ones_n = jnp.ones((NP,), jnp.float32)
    idxcol = jnp.arange(NP, dtype=jnp.int32).reshape(NP, 1)
    xp = jnp.pad(x, ((0, NP - N), (0, 0)))
    b1 = conv1_bl.reshape(1, D)
    b2 = conv2_bl.reshape(1, D)
    b3 = conv3_bl.reshape(1, D)
    b4 = conv4_bl.reshape(1, D)
    Wp1 = jnp.concatenate([pool1_Wrel, pool1_Wroot], axis=1)
    Wp2 = jnp.concatenate([pool2_Wrel, pool2_Wroot], axis=1)

    cnt1T = _sc_scalar(ones_n, src2, dst2)
    A1 = _sc_rows(x, src3, dst3)
    h1 = _tc_conv_call(A1, cnt1T, xp, conv1_Wl, b1, conv1_Wr)

    A2 = _sc_rows(h1, src3, dst3)
    h2, pools1 = _tc_conv_call(A2, cnt1T, h1, conv2_Wl, b2, conv2_Wr, Wp=Wp1)
    r1 = pools1[:, 0]
    root1 = pools1[:, 1:2]

    S1T = _sc_scalar(r1, src2, dst2)
    act1, h2p = _tc_pool1_call(S1T, root1, pool1_brel.reshape(1, 1), h2,
                               idxcol, k=8000)

    cnt3T = _sc_scalar(act1.reshape(NP), src2, dst2)
    A3 = _sc_rows(h2p, src3, dst3)
    h3m = _tc_conv_call(A3, cnt3T, h2p, conv3_Wl, b3, conv3_Wr,
                        mask=act1, mask_h=True)

    A4 = _sc_rows(h3m, src3, dst3)
    h4, pools2 = _tc_conv_call(A4, cnt3T, h3m, conv4_Wl, b4, conv4_Wr,
                               mask=act1, Wp=Wp2, mask_r=True)
    r2m = pools2[:, 0]
    root2 = pools2[:, 1:2]

    S2T = _sc_scalar(r2m, src2, dst2)
    out = _tc_pool2_call(S2T, root2, pool2_brel.reshape(1, 1), act1, h4,
                         idxcol, fc1_W, fc1_b.reshape(1, 64),
                         fc2_W, fc2_b.reshape(1, 2), k=6400)
    return out.reshape(2)


# speedup vs baseline: 1.2038x; 1.0104x over previous
import jax
import jax.numpy as jnp
from jax import lax
from jax.experimental import pallas as pl
from jax.experimental.pallas import tpu as pltpu
from jax.experimental.pallas import tpu_sc as plsc

N = 10000
NP = 10240
E = 320000
D = 128
NW = 32
EPW = E // NW
CHK = 125
NCH = EPW // CHK
NHALF = 2
HNCH = NCH // NHALF
NBUF = 2
SPR = NP // 16
WCH = 80
BLK = 2048


def _sc_mesh():
    return plsc.VectorSubcoreMesh(core_axis_name="c", subcore_axis_name="s",
                                  num_cores=2, num_subcores=16)


def _sc_rows_body(table_hbm, src_hbm, dst_hbm, out_hbm,
                  src_v, dst_v, b0, b1, acc_sh, g0, g1):
    cid = lax.axis_index("c")
    sid = lax.axis_index("s")
    wid = cid * 16 + sid

    def zr(r, _):
        def zc(c, _):
            b0[r, pl.ds(c * 16, 16)] = jnp.zeros((16,), jnp.float32)
            return 0
        return lax.fori_loop(0, D // 16, zc, 0)
    lax.fori_loop(0, WCH, zr, 0)

    def zs(z, _):
        pltpu.sync_copy(b0.at[pl.ds(0, WCH)],
                        acc_sh.at[pl.ds(sid * SPR + z * WCH, WCH)])
        return 0
    lax.fori_loop(0, SPR // WCH, zs, 0)

    plsc.subcore_barrier()

    def half(h, _):
        pltpu.sync_copy(src_hbm.at[wid, h], src_v)
        pltpu.sync_copy(dst_hbm.at[wid, h], dst_v)
        pltpu.async_copy(table_hbm.at[src_v.at[0]], b0, g0)
        pltpu.async_copy(table_hbm.at[src_v.at[1]], b1, g1)

        def step(jj, _):
            j = 2 * jj
            pltpu.make_async_copy(table_hbm.at[src_v.at[j]], b0, g0).wait()
            pltpu.sync_copy(b0, acc_sh.at[dst_v.at[j]], add=True)
            pltpu.async_copy(table_hbm.at[src_v.at[j + 2]], b0, g0)
            pltpu.make_async_copy(table_hbm.at[src_v.at[j + 1]], b1,
                                  g1).wait()
            pltpu.sync_copy(b1, acc_sh.at[dst_v.at[j + 1]], add=True)
            pltpu.async_copy(table_hbm.at[src_v.at[j + 3]], b1, g1)
            return 0
        lax.fori_loop(0, HNCH // 2 - 1, step, 0)

        j = HNCH - 2
        pltpu.make_async_copy(table_hbm.at[src_v.at[j]], b0, g0).wait()
        pltpu.sync_copy(b0, acc_sh.at[dst_v.at[j]], add=True)
        pltpu.make_async_copy(table_hbm.at[src_v.at[j + 1]], b1, g1).wait()
        pltpu.sync_copy(b1, acc_sh.at[dst_v.at[j + 1]], add=True)
        return 0
    lax.fori_loop(0, NHALF, half, 0)
    plsc.subcore_barrier()

    def wout(z, _):
        base = sid * SPR + z * WCH
        wb = b0.at[pl.ds(0, WCH)]
        pltpu.sync_copy(acc_sh.at[pl.ds(base, WCH)], wb)
        pltpu.sync_copy(wb, out_hbm.at[cid].at[pl.ds(base, WCH)])
        return 0
    lax.fori_loop(0, SPR // WCH, wout, 0)


def _sc_rows(table, src3, dst3):
    f = pl.kernel(
        _sc_rows_body,
        out_type=jax.ShapeDtypeStruct((2, NP, D), jnp.float32),
        mesh=_sc_mesh(),
        scratch_types=[
            pltpu.VMEM((HNCH, CHK), jnp.int32),
            pltpu.VMEM((HNCH, CHK), jnp.int32),
            pltpu.VMEM((CHK, D), jnp.float32),
            pltpu.VMEM((CHK, D), jnp.float32),
            pltpu.VMEM_SHARED((NP, D), jnp.float32),
            pltpu.SemaphoreType.DMA,
            pltpu.SemaphoreType.DMA,
        ],
    )
    return f(table, src3, dst3)


def _sc_scalar_body(vals_hbm, src_hbm, dst_hbm, out_hbm,
                    tab_v, src_v, dst_v, acc_v):
    cid = lax.axis_index("c")
    sid = lax.axis_index("s")
    wid = cid * 16 + sid
    pltpu.sync_copy(vals_hbm, tab_v)
    pltpu.sync_copy(src_hbm.at[wid], src_v)
    pltpu.sync_copy(dst_hbm.at[wid], dst_v)

    def z(i, _):
        acc_v[pl.ds(i * 16, 16)] = jnp.zeros((16,), jnp.float32)
        return 0
    lax.fori_loop(0, NP // 16, z, 0)

    def step(i, _):
        s = src_v[0, pl.ds(i * 16, 16)]
        t = dst_v[0, pl.ds(i * 16, 16)]
        v = plsc.load_gather(tab_v, [s])
        plsc.addupdate_scatter(acc_v, [t], v)
        return 0
    lax.fori_loop(0, EPW // 16, step, 0)
    pltpu.sync_copy(acc_v, out_hbm.at[wid, 0])


def _sc_scalar(vals, src2, dst2):
    f = pl.kernel(
        _sc_scalar_body,
        out_type=jax.ShapeDtypeStruct((NW, 1, NP), jnp.float32),
        mesh=_sc_mesh(),
        compiler_params=pltpu.CompilerParams(needs_layout_passes=False),
        scratch_types=[
            pltpu.VMEM((NP,), jnp.float32),
            pltpu.VMEM((1, EPW), jnp.int32),
            pltpu.VMEM((1, EPW), jnp.int32),
            pltpu.VMEM((NP,), jnp.float32),
        ],
    )
    return f(vals, src2, dst2).reshape(NW, NP).T


def _tc_conv_call(parts, cntT, xin, Wl, bl, Wr, mask=None, Wp=None,
                  mask_h=False, mask_r=False):
    has_mask = mask is not None
    has_pool = Wp is not None

    def body(*refs):
        i = 0
        parts_r = refs[i]; i += 1
        cnt_r = refs[i]; i += 1
        x_r = refs[i]; i += 1
        wl_r = refs[i]; i += 1
        bl_r = refs[i]; i += 1
        wr_r = refs[i]; i += 1
        mask_ref = refs[i] if has_mask else None
        i += 1 if has_mask else 0
        wp_r = refs[i] if has_pool else None
        i += 1 if has_pool else 0
        h_out = refs[i]; i += 1
        pool_out = refs[i] if has_pool else None

        A = parts_r[0] + parts_r[1]
        cnt = jnp.sum(cnt_r[...], axis=1, keepdims=True)
        mean = A * (1.0 / jnp.maximum(cnt, 1.0))
        h = jnp.dot(mean, wl_r[...], preferred_element_type=jnp.float32)
        h = h + bl_r[...] + jnp.dot(x_r[...], wr_r[...],
                                    preferred_element_type=jnp.float32)
        h = jnp.maximum(h, 0.0)
        if mask_h:
            h = h * mask_ref[...]
        h_out[...] = h
        if has_pool:
            p = jnp.dot(h, wp_r[...], preferred_element_type=jnp.float32)
            if mask_r:
                p = p * jnp.concatenate(
                    [mask_ref[...], jnp.ones_like(mask_ref[...])], axis=1)
            pool_out[...] = p

    grid = NP // BLK
    in_specs = [
        pl.BlockSpec((2, BLK, D), lambda i: (0, i, 0)),
        pl.BlockSpec((BLK, NW), lambda i: (i, 0)),
        pl.BlockSpec((BLK, D), lambda i: (i, 0)),
        pl.BlockSpec((D, D), lambda i: (0, 0)),
        pl.BlockSpec((1, D), lambda i: (0, 0)),
        pl.BlockSpec((D, D), lambda i: (0, 0)),
    ]
    args = [parts, cntT, xin, Wl, bl, Wr]
    if has_mask:
        in_specs.append(pl.BlockSpec((BLK, 1), lambda i: (i, 0)))
        args.append(mask)
    if has_pool:
        in_specs.append(pl.BlockSpec((D, 2), lambda i: (0, 0)))
        args.append(Wp)
    out_shape = [jax.ShapeDtypeStruct((NP, D), jnp.float32)]
    out_specs = [pl.BlockSpec((BLK, D), lambda i: (i, 0))]
    if has_pool:
        out_shape.append(jax.ShapeDtypeStruct((NP, 2), jnp.float32))
        out_specs.append(pl.BlockSpec((BLK, 2), lambda i: (i, 0)))

    res = pl.pallas_call(
        body,
        grid=(grid,),
        in_specs=in_specs,
        out_specs=out_specs,
        out_shape=out_shape,
    )(*args)
    return res if has_pool else res[0]


def _topk_mask(score, idx, k):
    key = lax.bitcast_convert_type(score, jnp.int32)
    ukey = (jnp.where(key < 0, jnp.int32(0x7FFFFFFF) ^ key, key)
            .astype(jnp.uint32) + jnp.uint32(2 ** 31))

    def tstep(b, t):
        cand = t | (jnp.uint32(1) << (31 - b))
        c = jnp.sum((ukey >= cand).astype(jnp.int32))
        return jnp.where(c >= k, cand, t)
    t = lax.fori_loop(0, 32, tstep, jnp.uint32(0))

    cnt_gt = jnp.sum((ukey > t).astype(jnp.int32))
    need = k - cnt_gt

    def mstep(b, m):
        cand = m | (jnp.int32(1) << (13 - b))
        c = jnp.sum(((ukey == t) & (idx < cand)).astype(jnp.int32))
        return jnp.where(c < need, cand, m)
    m = lax.fori_loop(0, 14, mstep, jnp.int32(0))

    act = (ukey > t) | ((ukey == t) & (idx <= m))
    return act.astype(jnp.float32)


def _tc_pool1_call(spartsT, root, brel, h2, idxcol, k):
    def body(sp_r, root_r, brel_r, h2_r, idx_r, act_out, hp_out):
        S = jnp.sum(sp_r[...], axis=1, keepdims=True)
        score = S + brel_r[0, 0] + root_r[...]
        score = jnp.where(idx_r[...] < N, score, -jnp.inf)
        act = _topk_mask(score, idx_r[...], k)
        act_out[...] = act
        hp_out[...] = h2_r[...] * (jnp.tanh(score) * act)

    return pl.pallas_call(
        body,
        out_shape=[jax.ShapeDtypeStruct((NP, 1), jnp.float32),
                   jax.ShapeDtypeStruct((NP, D), jnp.float32)],
    )(spartsT, root, brel, h2, idxcol)


def _tc_pool2_call(spartsT, root, brel, act1, h4, idxcol,
                   f1W, f1b, f2W, f2b, k):
    def body(sp_r, root_r, brel_r, act1_r, h4_r, idx_r,
             f1w_r, f1b_r, f2w_r, f2b_r, out_r):
        S = jnp.sum(sp_r[...], axis=1, keepdims=True)
        score = S + brel_r[0, 0] + root_r[...]
        score = jnp.where(act1_r[...] > 0, score, -jnp.inf)
        act2 = _topk_mask(score, idx_r[...], k)
        gate = jnp.tanh(score) * act2
        g = jnp.sum(h4_r[...] * gate, axis=0, keepdims=True) / float(k)
        g = jnp.dot(g, f1w_r[...], preferred_element_type=jnp.float32)
        g = jnp.maximum(g + f1b_r[...], 0.0)
        o = jnp.dot(g, f2w_r[...], preferred_element_type=jnp.float32)
        o = o + f2b_r[...]
        omax = jnp.max(o)
        lse = jnp.log(jnp.sum(jnp.exp(o - omax))) + omax
        out_r[...] = o - lse

    return pl.pallas_call(
        body,
        out_shape=jax.ShapeDtypeStruct((1, 2), jnp.float32),
    )(spartsT, root, brel, act1, h4, idxcol, f1W, f1b, f2W, f2b)


def kernel(x, edge_index, batch,
           conv1_Wl, conv1_bl, conv1_Wr, conv2_Wl, conv2_bl, conv2_Wr,
           conv3_Wl, conv3_bl, conv3_Wr, conv4_Wl, conv4_bl, conv4_Wr,
           pool1_Wrel, pool1_brel, pool1_Wroot,
           pool2_Wrel, pool2_brel, pool2_Wroot,
           fc1_W, fc1_b, fc2_W, fc2_b):
    src = edge_index[0]
    dst = edge_index[1]
    src3 = src.reshape(NW, NHALF, HNCH, CHK)
    dst3 = dst.reshape(NW, NHALF, HNCH, CHK)
    src2 = src.reshape(NW, 1, EPW)
    dst2 = dst.reshape(NW, 1, EPW)
    ones_n = jnp.ones((NP,), jnp.float32)
    idxcol = jnp.arange(NP, dtype=jnp.int32).reshape(NP, 1)
    xp = jnp.pad(x, ((0, NP - N), (0, 0)))
    b1 = conv1_bl.reshape(1, D)
    b2 = conv2_bl.reshape(1, D)
    b3 = conv3_bl.reshape(1, D)
    b4 = conv4_bl.reshape(1, D)
    Wp1 = jnp.concatenate([pool1_Wrel, pool1_Wroot], axis=1)
    Wp2 = jnp.concatenate([pool2_Wrel, pool2_Wroot], axis=1)

    cnt1T = _sc_scalar(ones_n, src2, dst2)
    A1 = _sc_rows(x, src3, dst3)
    h1 = _tc_conv_call(A1, cnt1T, xp, conv1_Wl, b1, conv1_Wr)

    A2 = _sc_rows(h1, src3, dst3)
    h2, pools1 = _tc_conv_call(A2, cnt1T, h1, conv2_Wl, b2, conv2_Wr, Wp=Wp1)
    r1 = pools1[:, 0]
    root1 = pools1[:, 1:2]

    S1T = _sc_scalar(r1, src2, dst2)
    act1, h2p = _tc_pool1_call(S1T, root1, pool1_brel.reshape(1, 1), h2,
                               idxcol, k=8000)

    cnt3T = _sc_scalar(act1.reshape(NP), src2, dst2)
    A3 = _sc_rows(h2p, src3, dst3)
    h3m = _tc_conv_call(A3, cnt3T, h2p, conv3_Wl, b3, conv3_Wr,
                        mask=act1, mask_h=True)

    A4 = _sc_rows(h3m, src3, dst3)
    h4, pools2 = _tc_conv_call(A4, cnt3T, h3m, conv4_Wl, b4, conv4_Wr,
                               mask=act1, Wp=Wp2, mask_r=True)
    r2m = pools2[:, 0]
    root2 = pools2[:, 1:2]

    S2T = _sc_scalar(r2m, src2, dst2)
    out = _tc_pool2_call(S2T, root2, pool2_brel.reshape(1, 1), act1, h4,
                         idxcol, fc1_W, fc1_b.reshape(1, 64),
                         fc2_W, fc2_b.reshape(1, 2), k=6400)
    return out.reshape(2)
